# Initial kernel scaffold; baseline (speedup 1.0000x reference)
import jax
import jax.numpy as jnp
from jax.experimental import pallas as pl


def kernel(x_nchw, weight, gamma, beta):
    raise NotImplementedError("write your pallas kernel here")


# speedup vs baseline: 1.6292x; 1.6292x over previous
import jax
import jax.numpy as jnp
from jax import lax
from jax.experimental import pallas as pl
from jax.experimental.pallas import tpu as pltpu

_EPS = 1e-5


def _stats_body(x_ref, g_ref, s_ref):
    @pl.when(pl.program_id(1) == 0)
    def _():
        g_ref[...] = jnp.zeros_like(g_ref)
        s_ref[...] = jnp.zeros_like(s_ref)

    x = x_ref[...]
    xb = x.astype(jnp.bfloat16)
    g_ref[...] += lax.dot_general(
        xb, xb, (((1,), (1,)), ((), ())),
        preferred_element_type=jnp.float32)
    s_ref[...] += jnp.sum(x, axis=1, keepdims=True)


def _apply_body(w_ref, shift_ref, x_ref, o_ref):
    xb = x_ref[...].astype(jnp.bfloat16)
    y = jnp.dot(w_ref[...], xb, preferred_element_type=jnp.float32)
    o_ref[...] = jnp.maximum(y + shift_ref[...], 0.0)


def kernel(x_nchw, weight, gamma, beta):
    N, C_in, H, W = x_nchw.shape
    C_out = weight.shape[0]
    HW = H * W
    M = float(N * HW)
    x3d = x_nchw.reshape(N, C_in, HW)

    n_split = 2 if N % 2 == 0 else 1
    nb = N // n_split
    x_bytes = 4 * N * C_in * HW

    g_parts, s_parts = pl.pallas_call(
        _stats_body,
        out_shape=(jax.ShapeDtypeStruct((n_split, C_in, C_in), jnp.float32),
                   jax.ShapeDtypeStruct((n_split, C_in, 1), jnp.float32)),
        grid=(n_split, nb),
        in_specs=[pl.BlockSpec((None, C_in, HW), lambda c, b: (c * nb + b, 0, 0))],
        out_specs=(pl.BlockSpec((None, C_in, C_in), lambda c, b: (c, 0, 0)),
                   pl.BlockSpec((None, C_in, 1), lambda c, b: (c, 0, 0))),
        compiler_params=pltpu.CompilerParams(
            dimension_semantics=("parallel", "arbitrary")),
        cost_estimate=pl.CostEstimate(
            flops=2 * N * HW * C_in * C_in, transcendentals=0,
            bytes_accessed=x_bytes + 4 * n_split * (C_in * C_in + C_in)),
    )(x3d)
    g = g_parts.sum(axis=0)
    s = s_parts.sum(axis=0)

    w = weight.astype(jnp.float32)
    mean = (w @ s) / M
    e_y2 = jnp.sum((w @ g) * w, axis=1, keepdims=True) / M
    var = jnp.maximum(e_y2 - mean * mean, 0.0)
    inv = lax.rsqrt(var + _EPS)
    scale = gamma.reshape(C_out, 1).astype(jnp.float32) * inv
    shift = beta.reshape(C_out, 1).astype(jnp.float32) - mean * scale
    w_fold = (scale * w).astype(jnp.bfloat16)

    out3d = pl.pallas_call(
        _apply_body,
        out_shape=jax.ShapeDtypeStruct((N, C_out, HW), jnp.float32),
        grid=(N,),
        in_specs=[
            pl.BlockSpec((C_out, C_in), lambda b: (0, 0)),
            pl.BlockSpec((C_out, 1), lambda b: (0, 0)),
            pl.BlockSpec((None, C_in, HW), lambda b: (b, 0, 0)),
        ],
        out_specs=pl.BlockSpec((None, C_out, HW), lambda b: (b, 0, 0)),
        compiler_params=pltpu.CompilerParams(
            dimension_semantics=("parallel",)),
        cost_estimate=pl.CostEstimate(
            flops=2 * N * HW * C_in * C_out, transcendentals=0,
            bytes_accessed=x_bytes + 4 * N * C_out * HW + 2 * C_out * (C_in + 2)),
    )(w_fold, shift, x3d)

    return out3d.reshape(N, C_out, H, W)


# speedup vs baseline: 1.8401x; 1.1295x over previous
import jax
import jax.numpy as jnp
from jax import lax
from jax.experimental import pallas as pl
from jax.experimental.pallas import tpu as pltpu

_EPS = 1e-5


def _fused_body(w_ref, gamma_ref, beta_ref, x_ref, o_ref,
                xbf_ref, g_ref, s_ref, wf_ref, shift_ref, *, n, m_true):
    p = pl.program_id(0)
    b = pl.program_id(1)
    c_in = x_ref.shape[0]

    @pl.when((p == 0) & (b == 0))
    def _():
        g_ref[...] = jnp.zeros_like(g_ref)
        s_ref[...] = jnp.zeros_like(s_ref)

    @pl.when(p == 0)
    def _():
        x = x_ref[...]
        xb = x.astype(jnp.bfloat16)
        xbf_ref[b] = xb
        g_ref[...] += lax.dot_general(
            xb, xb, (((1,), (1,)), ((), ())),
            preferred_element_type=jnp.float32)
        s_ref[...] += jnp.sum(x, axis=1, keepdims=True)

    @pl.when((p == 0) & (b == n - 1))
    def _():
        w = w_ref[...].astype(jnp.float32)
        g = g_ref[...]
        s = s_ref[...]
        ws = jnp.dot(w, jnp.broadcast_to(s, (c_in, c_in)),
                     preferred_element_type=jnp.float32)[:, :1]
        mean = ws / m_true
        wg = jnp.dot(w, g, preferred_element_type=jnp.float32)
        e_y2 = jnp.sum(wg * w, axis=1, keepdims=True) / m_true
        var = jnp.maximum(e_y2 - mean * mean, 0.0)
        inv = lax.rsqrt(var + _EPS)
        scale = gamma_ref[...] * inv
        shift_ref[...] = beta_ref[...] - mean * scale
        wf_ref[...] = (scale * w).astype(jnp.bfloat16)

    @pl.when(p == 1)
    def _():
        y = jnp.dot(wf_ref[...], xbf_ref[b],
                    preferred_element_type=jnp.float32)
        o_ref[...] = jnp.maximum(y + shift_ref[...], 0.0)


def kernel(x_nchw, weight, gamma, beta):
    N, C_in, H, W = x_nchw.shape
    C_out = weight.shape[0]
    HW = H * W
    M = float(N * HW)
    x3d = x_nchw.reshape(N, C_in, HW)
    g2 = gamma.reshape(C_out, 1).astype(jnp.float32)
    b2 = beta.reshape(C_out, 1).astype(jnp.float32)

    import functools
    body = functools.partial(_fused_body, n=N, m_true=M)

    out3d = pl.pallas_call(
        body,
        out_shape=jax.ShapeDtypeStruct((N, C_out, HW), jnp.float32),
        grid=(2, N),
        in_specs=[
            pl.BlockSpec((C_out, C_in), lambda p, b: (0, 0)),
            pl.BlockSpec((C_out, 1), lambda p, b: (0, 0)),
            pl.BlockSpec((C_out, 1), lambda p, b: (0, 0)),
            pl.BlockSpec((None, C_in, HW),
                         lambda p, b: (b + p * (N - 1 - b), 0, 0)),
        ],
        out_specs=pl.BlockSpec((None, C_out, HW),
                               lambda p, b: (p * b, 0, 0)),
        scratch_shapes=[
            pltpu.VMEM((N, C_in, HW), jnp.bfloat16),
            pltpu.VMEM((C_in, C_in), jnp.float32),
            pltpu.VMEM((C_in, 1), jnp.float32),
            pltpu.VMEM((C_out, C_in), jnp.bfloat16),
            pltpu.VMEM((C_out, 1), jnp.float32),
        ],
        compiler_params=pltpu.CompilerParams(
            dimension_semantics=("arbitrary", "arbitrary")),
        cost_estimate=pl.CostEstimate(
            flops=2 * N * HW * C_in * (C_in + C_out), transcendentals=C_out,
            bytes_accessed=4 * N * HW * (C_in + C_out)),
    )(weight, g2, b2, x3d)

    return out3d.reshape(N, C_out, H, W)


# speedup vs baseline: 1.9342x; 1.0511x over previous
import jax
import jax.numpy as jnp
from jax import lax
from jax.experimental import pallas as pl
from jax.experimental.pallas import tpu as pltpu

_EPS = 1e-5


def _fused_body(w_ref, gamma_ref, beta_ref, x_ref, o_ref,
                xbf_ref, g_ref, s_ref, wf_ref, shift_ref, *, n, nb, m_true):
    p = pl.program_id(0)
    b = pl.program_id(1)
    c_in = x_ref.shape[1]

    @pl.when((p == 0) & (b == 0))
    def _():
        g_ref[...] = jnp.zeros_like(g_ref)
        s_ref[...] = jnp.zeros_like(s_ref)

    @pl.when(p == 0)
    def _():
        for i in range(nb):
            x = x_ref[i]
            xb = x.astype(jnp.bfloat16)
            xbf_ref[b * nb + i] = xb
            g_ref[...] += lax.dot_general(
                xb, xb, (((1,), (1,)), ((), ())),
                preferred_element_type=jnp.float32)
            s_ref[...] += jnp.sum(x, axis=1, keepdims=True)

    @pl.when((p == 0) & (b == n // nb - 1))
    def _():
        w = w_ref[...].astype(jnp.float32)
        g = g_ref[...]
        s = s_ref[...]
        ws = jnp.dot(w, jnp.broadcast_to(s, (c_in, c_in)),
                     preferred_element_type=jnp.float32)[:, :1]
        mean = ws / m_true
        wg = jnp.dot(w, g, preferred_element_type=jnp.float32)
        e_y2 = jnp.sum(wg * w, axis=1, keepdims=True) / m_true
        var = jnp.maximum(e_y2 - mean * mean, 0.0)
        inv = lax.rsqrt(var + _EPS)
        scale = gamma_ref[...] * inv
        shift_ref[...] = beta_ref[...] - mean * scale
        wf_ref[...] = (scale * w).astype(jnp.bfloat16)

    @pl.when(p == 1)
    def _():
        for i in range(nb):
            y = jnp.dot(wf_ref[...], xbf_ref[b * nb + i],
                        preferred_element_type=jnp.float32)
            o_ref[i] = jnp.maximum(y + shift_ref[...], 0.0)


def kernel(x_nchw, weight, gamma, beta):
    N, C_in, H, W = x_nchw.shape
    C_out = weight.shape[0]
    HW = H * W
    M = float(N * HW)
    x3d = x_nchw.reshape(N, C_in, HW)
    g2 = gamma.reshape(C_out, 1).astype(jnp.float32)
    b2 = beta.reshape(C_out, 1).astype(jnp.float32)

    import functools
    NB = 2 if N % 2 == 0 else 1
    NP = N // NB
    body = functools.partial(_fused_body, n=N, nb=NB, m_true=M)
    x4d = x3d.reshape(NP, NB, C_in, HW)

    out4d = pl.pallas_call(
        body,
        out_shape=jax.ShapeDtypeStruct((NP, NB, C_out, HW), jnp.float32),
        grid=(2, NP),
        in_specs=[
            pl.BlockSpec((C_out, C_in), lambda p, b: (0, 0)),
            pl.BlockSpec((C_out, 1), lambda p, b: (0, 0)),
            pl.BlockSpec((C_out, 1), lambda p, b: (0, 0)),
            pl.BlockSpec((None, NB, C_in, HW),
                         lambda p, b: (b + p * (NP - 1 - b), 0, 0, 0)),
        ],
        out_specs=pl.BlockSpec((None, NB, C_out, HW),
                               lambda p, b: (p * b, 0, 0, 0)),
        scratch_shapes=[
            pltpu.VMEM((N, C_in, HW), jnp.bfloat16),
            pltpu.VMEM((C_in, C_in), jnp.float32),
            pltpu.VMEM((C_in, 1), jnp.float32),
            pltpu.VMEM((C_out, C_in), jnp.bfloat16),
            pltpu.VMEM((C_out, 1), jnp.float32),
        ],
        compiler_params=pltpu.CompilerParams(
            dimension_semantics=("arbitrary", "arbitrary")),
        cost_estimate=pl.CostEstimate(
            flops=2 * N * HW * C_in * (C_in + C_out), transcendentals=C_out,
            bytes_accessed=4 * N * HW * (C_in + C_out)),
    )(weight, g2, b2, x4d)

    return out4d.reshape(N, C_out, H, W)
